# baseline (speedup 1.0000x reference)
import functools

import jax
import jax.numpy as jnp
from jax import lax
from jax.experimental import pallas as pl
from jax.experimental.pallas import tpu as pltpu
from jax.experimental.pallas import tpu_sc as plsc

C = 128
K = 16
N = 10000
BLK = 400
SCALE = 30.0
EPS = 1e-5

_SC_CORES = 2
_SC_SUBCORES = 16
_NW = _SC_CORES * _SC_SUBCORES
_CHUNK = 128
_NBUF = 2
_Q = 5


def _gelu(x):
    return jax.nn.gelu(x)


def _ln(x, g, b):
    mu = jnp.mean(x, axis=-1, keepdims=True)
    var = jnp.mean((x - mu) ** 2, axis=-1, keepdims=True)
    return (x - mu) / jnp.sqrt(var + EPS) * g + b


def _proj_body(x_ref, w_ref, o_ref):
    o_ref[...] = jnp.dot(x_ref[...].astype(jnp.bfloat16), w_ref[...],
                         preferred_element_type=jnp.float32)


def _project(x, w):
    n, _ = x.shape
    d = w.shape[1]
    return pl.pallas_call(
        _proj_body,
        out_shape=jax.ShapeDtypeStruct((n, d), jnp.float32),
    )(x, w)


def _sc_gather(table, idx):
    v, d = table.shape
    r = idx.shape[0]
    per_w = r // _NW
    sizes = [_CHUNK] * (per_w // _CHUNK)
    if per_w % _CHUNK:
        sizes.append(per_w % _CHUNK)
    offs = [sum(sizes[:i]) for i in range(len(sizes))]
    mesh = plsc.VectorSubcoreMesh(
        core_axis_name="c", subcore_axis_name="s",
        num_cores=_SC_CORES, num_subcores=_SC_SUBCORES)

    @functools.partial(
        pl.kernel, mesh=mesh,
        out_type=jax.ShapeDtypeStruct((r, d), jnp.float32),
        scratch_types=(
            [pltpu.VMEM((per_w,), jnp.int32)]
            + [pltpu.VMEM((_CHUNK, d), jnp.float32)] * _NBUF
            + [pltpu.SemaphoreType.DMA] * (2 * _NBUF)
        ),
    )
    def k(table_hbm, idx_hbm, out_hbm, idx_v, *bufs_sems):
        bufs = bufs_sems[:_NBUF]
        gsems = bufs_sems[_NBUF:2 * _NBUF]
        wsems = bufs_sems[2 * _NBUF:]
        wid = lax.axis_index("s") * _SC_CORES + lax.axis_index("c")
        base = wid * per_w
        pltpu.sync_copy(idx_hbm.at[pl.ds(base, per_w)], idx_v)

        def gcopy(c, b):
            return pltpu.make_async_copy(
                table_hbm.at[idx_v.at[pl.ds(offs[c], sizes[c])]],
                bufs[b].at[pl.ds(0, sizes[c])], gsems[b])

        def wcopy(c, b):
            return pltpu.make_async_copy(
                bufs[b].at[pl.ds(0, sizes[c])],
                out_hbm.at[pl.ds(base + offs[c], sizes[c])], wsems[b])

        nch = len(sizes)
        for b in range(min(_NBUF, nch)):
            gcopy(b, b).start()
        for c in range(nch):
            b = c % _NBUF
            gcopy(c, b).wait()
            wcopy(c, b).start()
            nxt = c + _NBUF
            if nxt < nch:
                wcopy(c, b).wait()
                gcopy(nxt, b).start()
        for c in range(max(0, nch - _NBUF), nch):
            wcopy(c, c % _NBUF).wait()

    return k(table, idx)


def _bdot(x, w):
    return jnp.dot(x.astype(jnp.bfloat16), w,
                   preferred_element_type=jnp.float32)


def _pass1_body(hv_ref, he_ref, g_ref, ma_ref, mask_ref,
                w1a_ref, w1b_ref, b1_ref, w2_ref, b2_ref, w3_ref, b3_ref,
                n1g_ref, n1b_ref, wf1_ref, bf1_ref, wf2_ref, bf2_ref,
                n2g_ref, n2b_ref, wp_ref,
                hv1_ref, t1_ref):
    hv = hv_ref[...]
    he = he_ref[...]
    a = _bdot(hv, w1a_ref[...]) + b1_ref[...]
    e = jnp.dot(he, w1b_ref[...],
                preferred_element_type=jnp.float32)
    m = (e + g_ref[...]).reshape(BLK, K, C) + a[:, None, :]
    m = _gelu(m.astype(jnp.bfloat16)).reshape(BLK * K, C)
    m = jnp.dot(m, w2_ref[...], preferred_element_type=jnp.float32)
    m = _gelu((m + b2_ref[...]).astype(jnp.bfloat16))
    m = jnp.dot(m, w3_ref[...],
                preferred_element_type=jnp.float32) + b3_ref[...]
    m = m * ma_ref[...]
    dh = jnp.sum(m.reshape(BLK, K, C), axis=1) * (1.0 / SCALE)
    h = _ln(hv + dh, n1g_ref[...], n1b_ref[...])
    t = _bdot(h, wf1_ref[...]) + bf1_ref[...]
    t = _gelu(t.astype(jnp.bfloat16))
    dh2 = jnp.dot(t, wf2_ref[...],
                  preferred_element_type=jnp.float32) + bf2_ref[...]
    h = _ln(h + dh2, n2g_ref[...], n2b_ref[...])
    h = h * mask_ref[...]
    hv1_ref[...] = h
    t1_ref[...] = _bdot(h, wp_ref[...])


def _pass2_body(hv_ref, he_ref, g_ref, ma_ref, mask_ref,
                w11a_ref, w11b_ref, b11_ref, w12_ref, b12_ref,
                w13_ref, b13_ref, n3g_ref, n3b_ref,
                w1a_ref, w1b_ref, b1_ref, w2_ref, b2_ref, w3_ref, b3_ref,
                n1g_ref, n1b_ref, wf1_ref, bf1_ref, wf2_ref, bf2_ref,
                n2g_ref, n2b_ref,
                hv2_ref):
    hv = hv_ref[...]
    he = he_ref[...]
    g = g_ref[...]

    ae = _bdot(hv, w11a_ref[...]) + b11_ref[...]
    me = jnp.dot(he, w11b_ref[...], preferred_element_type=jnp.float32)
    me = (me + g[:, :C]).reshape(BLK, K, C) + ae[:, None, :]
    me = _gelu(me.astype(jnp.bfloat16)).reshape(BLK * K, C)
    me = jnp.dot(me, w12_ref[...], preferred_element_type=jnp.float32)
    me = _gelu((me + b12_ref[...]).astype(jnp.bfloat16))
    me = jnp.dot(me, w13_ref[...],
                 preferred_element_type=jnp.float32) + b13_ref[...]
    he1 = _ln(he.astype(jnp.float32) + me, n3g_ref[...], n3b_ref[...])

    a = _bdot(hv, w1a_ref[...]) + b1_ref[...]
    m = _bdot(he1, w1b_ref[...])
    m = (m + g[:, C:]).reshape(BLK, K, C) + a[:, None, :]
    m = _gelu(m.astype(jnp.bfloat16)).reshape(BLK * K, C)
    m = jnp.dot(m, w2_ref[...], preferred_element_type=jnp.float32)
    m = _gelu((m + b2_ref[...]).astype(jnp.bfloat16))
    m = jnp.dot(m, w3_ref[...],
                preferred_element_type=jnp.float32) + b3_ref[...]
    m = m * ma_ref[...]
    dh = jnp.sum(m.reshape(BLK, K, C), axis=1) * (1.0 / SCALE)
    h = _ln(hv + dh, n1g_ref[...], n1b_ref[...])
    t = _bdot(h, wf1_ref[...]) + bf1_ref[...]
    t = _gelu(t.astype(jnp.bfloat16))
    dh2 = jnp.dot(t, wf2_ref[...],
                  preferred_element_type=jnp.float32) + bf2_ref[...]
    h = _ln(h + dh2, n2g_ref[...], n2b_ref[...])
    hv2_ref[...] = h * mask_ref[...]


def _full_spec(shape):
    nd = len(shape)
    return pl.BlockSpec(shape, lambda i: (0,) * nd)


def _node_spec(d, off=0):
    return pl.BlockSpec((BLK, d), lambda i, off=off: (off + i, 0))


def _edge_spec(d, off=0):
    return pl.BlockSpec((BLK * K, d), lambda i, off=off: (off + i, 0))


_QBLK = N // BLK // _Q


def _run_pass1(hv0, he0, g0, ma, msk, ws, q):
    off = q * _QBLK
    nq = N // _Q
    in_specs = ([_node_spec(C, off), _edge_spec(C, off), _edge_spec(C),
                 _edge_spec(1, off), _node_spec(1, off)]
                + [_full_spec(w.shape) for w in ws])
    return pl.pallas_call(
        _pass1_body,
        grid=(_QBLK,),
        in_specs=in_specs,
        out_specs=[_node_spec(C), _node_spec(2 * C)],
        out_shape=[jax.ShapeDtypeStruct((nq, C), jnp.float32),
                   jax.ShapeDtypeStruct((nq, 2 * C), jnp.float32)],
        compiler_params=pltpu.CompilerParams(
            dimension_semantics=("arbitrary",)),
    )(hv0, he0, g0, ma, msk, *ws)


def _run_pass2(hv1, he0, g1, ma, msk, ws, q):
    off = q * _QBLK
    nq = N // _Q
    in_specs = ([_node_spec(C, off), _edge_spec(C, off), _edge_spec(2 * C),
                 _edge_spec(1, off), _node_spec(1, off)]
                + [_full_spec(w.shape) for w in ws])
    return pl.pallas_call(
        _pass2_body,
        grid=(_QBLK,),
        in_specs=in_specs,
        out_specs=_node_spec(C),
        out_shape=jax.ShapeDtypeStruct((nq, C), jnp.float32),
        compiler_params=pltpu.CompilerParams(
            dimension_semantics=("arbitrary",)),
    )(hv1, he0, g1, ma, msk, *ws)


def kernel(h_V, h_E, E_idx, mask, mask_attend, params):
    b, n, c = h_V.shape
    k = E_idx.shape[2]
    hv0 = h_V.reshape(n, c)
    he0 = h_E.reshape(n * k, c).astype(jnp.bfloat16)
    idx = E_idx.reshape(n * k)
    msk = mask.reshape(n, 1)
    ma = mask_attend.reshape(n * k, 1)
    p0, p1 = params

    def row(x):
        return x.reshape(1, -1)

    def bw(x):
        return x.astype(jnp.bfloat16)

    rq = n * k // _Q

    table0 = _project(hv0, bw(p0['W1'][2 * c:, :]))
    g0s = [_sc_gather(table0, lax.dynamic_slice(idx, (q * rq,), (rq,)))
           for q in range(_Q)]
    wp = bw(jnp.concatenate([p0['W11'][2 * c:, :], p1['W1'][2 * c:, :]],
                            axis=1))
    ws1 = [bw(p0['W1'][:c, :]), bw(p0['W1'][c:2 * c, :]), row(p0['b1']),
           bw(p0['W2']), row(p0['b2']), bw(p0['W3']), row(p0['b3']),
           row(p0['n1g']), row(p0['n1b']),
           bw(p0['Wf1']), row(p0['bf1']), bw(p0['Wf2']), row(p0['bf2']),
           row(p0['n2g']), row(p0['n2b']), wp]
    p1_out = [_run_pass1(hv0, he0, g0s[q], ma, msk, ws1, q)
              for q in range(_Q)]
    hv1 = jnp.concatenate([o[0] for o in p1_out])
    t1 = jnp.concatenate([o[1] for o in p1_out])

    g1s = [_sc_gather(t1, lax.dynamic_slice(idx, (q * rq,), (rq,)))
           for q in range(_Q)]
    ws2 = [bw(p0['W11'][:c, :]), bw(p0['W11'][c:2 * c, :]), row(p0['b11']),
           bw(p0['W12']), row(p0['b12']), bw(p0['W13']), row(p0['b13']),
           row(p0['n3g']), row(p0['n3b']),
           bw(p1['W1'][:c, :]), bw(p1['W1'][c:2 * c, :]), row(p1['b1']),
           bw(p1['W2']), row(p1['b2']), bw(p1['W3']), row(p1['b3']),
           row(p1['n1g']), row(p1['n1b']),
           bw(p1['Wf1']), row(p1['bf1']), bw(p1['Wf2']), row(p1['bf2']),
           row(p1['n2g']), row(p1['n2b'])]
    hv2 = jnp.concatenate([_run_pass2(hv1, he0, g1s[q], ma, msk, ws2, q)
                           for q in range(_Q)])
    return hv2.reshape(b, n, c)


# speedup vs baseline: 1.0679x; 1.0679x over previous
import functools

import jax
import jax.numpy as jnp
from jax import lax
from jax.experimental import pallas as pl
from jax.experimental.pallas import tpu as pltpu
from jax.experimental.pallas import tpu_sc as plsc

C = 128
K = 16
N = 10000
BLK = 400
SCALE = 30.0
EPS = 1e-5

_SC_CORES = 2
_SC_SUBCORES = 16
_NW = _SC_CORES * _SC_SUBCORES
_CHUNK = 64
_NBUF = 4
_RPAD = 163840


def _gelu(x):
    return jax.nn.gelu(x)


def _ln(x, g, b):
    mu = jnp.mean(x, axis=-1, keepdims=True)
    xc = x - mu
    var = jnp.mean(xc * xc, axis=-1, keepdims=True)
    return xc * (lax.rsqrt(var + EPS) * g) + b


def _proj_body(x_ref, w_ref, o_ref):
    o_ref[...] = jnp.dot(x_ref[...].astype(jnp.bfloat16), w_ref[...],
                         preferred_element_type=jnp.float32)


def _project(x, w):
    n, _ = x.shape
    d = w.shape[1]
    return pl.pallas_call(
        _proj_body,
        out_shape=jax.ShapeDtypeStruct((n, d), jnp.float32),
    )(x, w)


def _sc_gather(table, idx):
    v, d = table.shape
    r = idx.shape[0]
    per_w = r // _NW
    nch = per_w // _CHUNK
    mesh = plsc.VectorSubcoreMesh(
        core_axis_name="c", subcore_axis_name="s",
        num_cores=_SC_CORES, num_subcores=_SC_SUBCORES)

    @functools.partial(
        pl.kernel, mesh=mesh,
        out_type=jax.ShapeDtypeStruct((r, d), jnp.float32),
        scratch_types=(
            [pltpu.VMEM((per_w,), jnp.int32)]
            + [pltpu.VMEM((_CHUNK, d), jnp.float32)] * _NBUF
            + [pltpu.SemaphoreType.DMA] * (2 * _NBUF)
        ),
    )
    def k(table_hbm, idx_hbm, out_hbm, idx_v, *bufs_sems):
        bufs = bufs_sems[:_NBUF]
        gsems = bufs_sems[_NBUF:2 * _NBUF]
        wsems = bufs_sems[2 * _NBUF:]
        wid = lax.axis_index("s") * _SC_CORES + lax.axis_index("c")
        base = wid * per_w
        pltpu.sync_copy(idx_hbm.at[pl.ds(base, per_w)], idx_v)

        def gcopy(c, b):
            return pltpu.make_async_copy(
                table_hbm.at[idx_v.at[pl.ds(c * _CHUNK, _CHUNK)]],
                bufs[b], gsems[b])

        def wcopy(c, b):
            return pltpu.make_async_copy(
                bufs[b], out_hbm.at[pl.ds(base + c * _CHUNK, _CHUNK)],
                wsems[b])

        for b in range(_NBUF):
            gcopy(b, b).start()

        @pl.loop(0, nch // _NBUF - 1)
        def _(t):
            c0 = t * _NBUF
            for b in range(_NBUF):
                gcopy(c0 + b, b).wait()
                wcopy(c0 + b, b).start()
            for b in range(_NBUF):
                wcopy(c0 + b, b).wait()
                gcopy(c0 + _NBUF + b, b).start()

        c0 = nch - _NBUF
        for b in range(_NBUF):
            gcopy(c0 + b, b).wait()
            wcopy(c0 + b, b).start()
        for b in range(_NBUF):
            wcopy(c0 + b, b).wait()

    return k(table, idx)


def _bdot(x, w):
    return jnp.dot(x.astype(jnp.bfloat16), w,
                   preferred_element_type=jnp.float32)


def _pass1_body(hv_ref, he_ref, g_ref, ma_ref, mask_ref,
                w1a_ref, w1b_ref, b1_ref, w2_ref, b2_ref, w3_ref, b3_ref,
                n1g_ref, n1b_ref, wf1_ref, bf1_ref, wf2_ref, bf2_ref,
                n2g_ref, n2b_ref, wp_ref,
                hv1_ref, t1_ref):
    hv = hv_ref[...]
    he = he_ref[...]
    a = _bdot(hv, w1a_ref[...]) + b1_ref[...]
    e = jnp.dot(he, w1b_ref[...],
                preferred_element_type=jnp.float32)
    m = (e + g_ref[...]).reshape(BLK, K, C) + a[:, None, :]
    m = _gelu(m.astype(jnp.bfloat16)).reshape(BLK * K, C)
    m = jnp.dot(m, w2_ref[...], preferred_element_type=jnp.float32)
    m = _gelu((m + b2_ref[...]).astype(jnp.bfloat16))
    m = jnp.dot(m, w3_ref[...],
                preferred_element_type=jnp.float32) + b3_ref[...]
    m = m * ma_ref[...]
    dh = jnp.sum(m.reshape(BLK, K, C), axis=1) * (1.0 / SCALE)
    h = _ln(hv + dh, n1g_ref[...], n1b_ref[...])
    t = _bdot(h, wf1_ref[...]) + bf1_ref[...]
    t = _gelu(t.astype(jnp.bfloat16))
    dh2 = jnp.dot(t, wf2_ref[...],
                  preferred_element_type=jnp.float32) + bf2_ref[...]
    h = _ln(h + dh2, n2g_ref[...], n2b_ref[...])
    h = h * mask_ref[...]
    hv1_ref[...] = h
    t1_ref[...] = _bdot(h, wp_ref[...])


def _pass2_body(hv_ref, he_ref, g_ref, ma_ref, mask_ref,
                w11a_ref, w11b_ref, b11_ref, w12_ref, b12_ref,
                w13_ref, b13_ref, n3g_ref, n3b_ref,
                w1a_ref, w1b_ref, b1_ref, w2_ref, b2_ref, w3_ref, b3_ref,
                n1g_ref, n1b_ref, wf1_ref, bf1_ref, wf2_ref, bf2_ref,
                n2g_ref, n2b_ref,
                hv2_ref):
    hv = hv_ref[...]
    he = he_ref[...]
    g = g_ref[...]

    ae = _bdot(hv, w11a_ref[...]) + b11_ref[...]
    me = jnp.dot(he, w11b_ref[...], preferred_element_type=jnp.float32)
    me = (me + g[:, :C]).reshape(BLK, K, C) + ae[:, None, :]
    me = _gelu(me.astype(jnp.bfloat16)).reshape(BLK * K, C)
    me = jnp.dot(me, w12_ref[...], preferred_element_type=jnp.float32)
    me = _gelu((me + b12_ref[...]).astype(jnp.bfloat16))
    me = jnp.dot(me, w13_ref[...],
                 preferred_element_type=jnp.float32) + b13_ref[...]
    he1 = _ln(he.astype(jnp.float32) + me, n3g_ref[...], n3b_ref[...])

    a = _bdot(hv, w1a_ref[...]) + b1_ref[...]
    m = _bdot(he1, w1b_ref[...])
    m = (m + g[:, C:]).reshape(BLK, K, C) + a[:, None, :]
    m = _gelu(m.astype(jnp.bfloat16)).reshape(BLK * K, C)
    m = jnp.dot(m, w2_ref[...], preferred_element_type=jnp.float32)
    m = _gelu((m + b2_ref[...]).astype(jnp.bfloat16))
    m = jnp.dot(m, w3_ref[...],
                preferred_element_type=jnp.float32) + b3_ref[...]
    m = m * ma_ref[...]
    dh = jnp.sum(m.reshape(BLK, K, C), axis=1) * (1.0 / SCALE)
    h = _ln(hv + dh, n1g_ref[...], n1b_ref[...])
    t = _bdot(h, wf1_ref[...]) + bf1_ref[...]
    t = _gelu(t.astype(jnp.bfloat16))
    dh2 = jnp.dot(t, wf2_ref[...],
                  preferred_element_type=jnp.float32) + bf2_ref[...]
    h = _ln(h + dh2, n2g_ref[...], n2b_ref[...])
    hv2_ref[...] = h * mask_ref[...]


def _full_spec(shape):
    nd = len(shape)
    return pl.BlockSpec(shape, lambda i: (0,) * nd)


def _node_spec(d):
    return pl.BlockSpec((BLK, d), lambda i: (i, 0))


def _edge_spec(d):
    return pl.BlockSpec((BLK * K, d), lambda i: (i, 0))


def _run_pass1(hv0, he0, g0, ma, msk, ws):
    in_specs = ([_node_spec(C), _edge_spec(C), _edge_spec(C),
                 _edge_spec(1), _node_spec(1)]
                + [_full_spec(w.shape) for w in ws])
    return pl.pallas_call(
        _pass1_body,
        grid=(N // BLK,),
        in_specs=in_specs,
        out_specs=[_node_spec(C), _node_spec(2 * C)],
        out_shape=[jax.ShapeDtypeStruct((N, C), jnp.float32),
                   jax.ShapeDtypeStruct((N, 2 * C), jnp.float32)],
        compiler_params=pltpu.CompilerParams(
            dimension_semantics=("arbitrary",)),
    )(hv0, he0, g0, ma, msk, *ws)


def _run_pass2(hv1, he0, g1, ma, msk, ws):
    in_specs = ([_node_spec(C), _edge_spec(C), _edge_spec(2 * C),
                 _edge_spec(1), _node_spec(1)]
                + [_full_spec(w.shape) for w in ws])
    return pl.pallas_call(
        _pass2_body,
        grid=(N // BLK,),
        in_specs=in_specs,
        out_specs=_node_spec(C),
        out_shape=jax.ShapeDtypeStruct((N, C), jnp.float32),
        compiler_params=pltpu.CompilerParams(
            dimension_semantics=("arbitrary",)),
    )(hv1, he0, g1, ma, msk, *ws)


def kernel(h_V, h_E, E_idx, mask, mask_attend, params):
    b, n, c = h_V.shape
    k = E_idx.shape[2]
    hv0 = h_V.reshape(n, c)
    he0 = h_E.reshape(n * k, c).astype(jnp.bfloat16)
    pad = jnp.arange(_RPAD - n * k, dtype=jnp.int32) % n
    idx = jnp.concatenate([E_idx.reshape(n * k), pad])
    msk = mask.reshape(n, 1)
    ma = mask_attend.reshape(n * k, 1)
    p0, p1 = params

    def row(x):
        return x.reshape(1, -1)

    def bw(x):
        return x.astype(jnp.bfloat16)

    table0 = _project(hv0, bw(p0['W1'][2 * c:, :]))
    g0 = _sc_gather(table0, idx)
    wp = bw(jnp.concatenate([p0['W11'][2 * c:, :], p1['W1'][2 * c:, :]],
                            axis=1))
    ws1 = [bw(p0['W1'][:c, :]), bw(p0['W1'][c:2 * c, :]), row(p0['b1']),
           bw(p0['W2']), row(p0['b2']), bw(p0['W3']), row(p0['b3']),
           row(p0['n1g']), row(p0['n1b']),
           bw(p0['Wf1']), row(p0['bf1']), bw(p0['Wf2']), row(p0['bf2']),
           row(p0['n2g']), row(p0['n2b']), wp]
    hv1, t1 = _run_pass1(hv0, he0, g0, ma, msk, ws1)

    g1 = _sc_gather(t1, idx)
    ws2 = [bw(p0['W11'][:c, :]), bw(p0['W11'][c:2 * c, :]), row(p0['b11']),
           bw(p0['W12']), row(p0['b12']), bw(p0['W13']), row(p0['b13']),
           row(p0['n3g']), row(p0['n3b']),
           bw(p1['W1'][:c, :]), bw(p1['W1'][c:2 * c, :]), row(p1['b1']),
           bw(p1['W2']), row(p1['b2']), bw(p1['W3']), row(p1['b3']),
           row(p1['n1g']), row(p1['n1b']),
           bw(p1['Wf1']), row(p1['bf1']), bw(p1['Wf2']), row(p1['bf2']),
           row(p1['n2g']), row(p1['n2b'])]
    hv2 = _run_pass2(hv1, he0, g1, ma, msk, ws2)
    return hv2.reshape(b, n, c)


# speedup vs baseline: 1.0705x; 1.0024x over previous
import functools

import jax
import jax.numpy as jnp
from jax import lax
from jax.experimental import pallas as pl
from jax.experimental.pallas import tpu as pltpu
from jax.experimental.pallas import tpu_sc as plsc

C = 128
K = 16
N = 10000
BLK = 400
SCALE = 30.0
EPS = 1e-5

_SC_CORES = 2
_SC_SUBCORES = 16
_NW = _SC_CORES * _SC_SUBCORES
_CHUNK = 64
_NBUF = 4
_RPAD = 163840


def _gelu(x):
    return jax.nn.gelu(x)


def _ln(x, g, b):
    mu = jnp.mean(x, axis=-1, keepdims=True)
    xc = x - mu
    var = jnp.mean(xc * xc, axis=-1, keepdims=True)
    return xc * (lax.rsqrt(var + EPS) * g) + b


def _proj_body(x_ref, w_ref, o_ref):
    o_ref[...] = jnp.dot(x_ref[...].astype(jnp.bfloat16), w_ref[...],
                         preferred_element_type=jnp.float32)


def _project(x, w):
    n, _ = x.shape
    d = w.shape[1]
    return pl.pallas_call(
        _proj_body,
        out_shape=jax.ShapeDtypeStruct((n, d), jnp.float32),
    )(x, w)


def _sc_gather(table, idx):
    v, d = table.shape
    r = idx.shape[0]
    per_w = r // _NW
    nch = per_w // _CHUNK
    mesh = plsc.VectorSubcoreMesh(
        core_axis_name="c", subcore_axis_name="s",
        num_cores=_SC_CORES, num_subcores=_SC_SUBCORES)

    @functools.partial(
        pl.kernel, mesh=mesh,
        out_type=jax.ShapeDtypeStruct((r, d), table.dtype),
        scratch_types=(
            [pltpu.VMEM((per_w,), jnp.int32)]
            + [pltpu.VMEM((_CHUNK, d), table.dtype)] * _NBUF
            + [pltpu.SemaphoreType.DMA] * (2 * _NBUF)
        ),
    )
    def k(table_hbm, idx_hbm, out_hbm, idx_v, *bufs_sems):
        bufs = bufs_sems[:_NBUF]
        gsems = bufs_sems[_NBUF:2 * _NBUF]
        wsems = bufs_sems[2 * _NBUF:]
        wid = lax.axis_index("s") * _SC_CORES + lax.axis_index("c")
        base = wid * per_w
        pltpu.sync_copy(idx_hbm.at[pl.ds(base, per_w)], idx_v)

        def gcopy(c, b):
            return pltpu.make_async_copy(
                table_hbm.at[idx_v.at[pl.ds(c * _CHUNK, _CHUNK)]],
                bufs[b], gsems[b])

        def wcopy(c, b):
            return pltpu.make_async_copy(
                bufs[b], out_hbm.at[pl.ds(base + c * _CHUNK, _CHUNK)],
                wsems[b])

        for b in range(_NBUF):
            gcopy(b, b).start()

        @pl.loop(0, nch // _NBUF - 1)
        def _(t):
            c0 = t * _NBUF
            for b in range(_NBUF):
                gcopy(c0 + b, b).wait()
                wcopy(c0 + b, b).start()
            for b in range(_NBUF):
                wcopy(c0 + b, b).wait()
                gcopy(c0 + _NBUF + b, b).start()

        c0 = nch - _NBUF
        for b in range(_NBUF):
            gcopy(c0 + b, b).wait()
            wcopy(c0 + b, b).start()
        for b in range(_NBUF):
            wcopy(c0 + b, b).wait()

    return k(table, idx)


def _bdot(x, w):
    return jnp.dot(x.astype(jnp.bfloat16), w,
                   preferred_element_type=jnp.float32)


def _pass1_body(hv_ref, he_ref, g_ref, ma_ref, mask_ref,
                w1a_ref, w1b_ref, b1_ref, w2_ref, b2_ref, w3_ref, b3_ref,
                n1g_ref, n1b_ref, wf1_ref, bf1_ref, wf2_ref, bf2_ref,
                n2g_ref, n2b_ref, wp_ref,
                hv1_ref, t1_ref):
    hv = hv_ref[...]
    he = he_ref[...]
    a = _bdot(hv, w1a_ref[...]) + b1_ref[...]
    e = jnp.dot(he, w1b_ref[...],
                preferred_element_type=jnp.float32)
    m = (e + g_ref[...]).reshape(BLK, K, C) + a[:, None, :]
    m = _gelu(m.astype(jnp.bfloat16)).reshape(BLK * K, C)
    m = jnp.dot(m, w2_ref[...], preferred_element_type=jnp.float32)
    m = _gelu(m.astype(jnp.bfloat16) + b2_ref[...])
    m = jnp.dot(m, w3_ref[...],
                preferred_element_type=jnp.float32) + b3_ref[...]
    m = m * ma_ref[...]
    dh = jnp.sum(m.reshape(BLK, K, C), axis=1) * (1.0 / SCALE)
    h = _ln(hv + dh, n1g_ref[...], n1b_ref[...])
    t = _bdot(h, wf1_ref[...])
    t = _gelu(t.astype(jnp.bfloat16) + bf1_ref[...])
    dh2 = jnp.dot(t, wf2_ref[...],
                  preferred_element_type=jnp.float32) + bf2_ref[...]
    h = _ln(h + dh2, n2g_ref[...], n2b_ref[...])
    h = h * mask_ref[...]
    hv1_ref[...] = h
    t1_ref[...] = _bdot(h, wp_ref[...])


def _pass2_body(hv_ref, he_ref, g_ref, ma_ref, mask_ref,
                w11a_ref, w11b_ref, b11_ref, w12_ref, b12_ref,
                w13_ref, b13_ref, n3g_ref, n3b_ref,
                w1a_ref, w1b_ref, b1_ref, w2_ref, b2_ref, w3_ref, b3_ref,
                n1g_ref, n1b_ref, wf1_ref, bf1_ref, wf2_ref, bf2_ref,
                n2g_ref, n2b_ref,
                hv2_ref):
    hv = hv_ref[...]
    he = he_ref[...]
    g = g_ref[...]

    ae = _bdot(hv, w11a_ref[...]) + b11_ref[...]
    me = jnp.dot(he, w11b_ref[...], preferred_element_type=jnp.float32)
    me = (me + g[:, :C]).reshape(BLK, K, C) + ae[:, None, :]
    me = _gelu(me.astype(jnp.bfloat16)).reshape(BLK * K, C)
    me = jnp.dot(me, w12_ref[...], preferred_element_type=jnp.float32)
    me = _gelu(me.astype(jnp.bfloat16) + b12_ref[...])
    me = jnp.dot(me, w13_ref[...],
                 preferred_element_type=jnp.float32) + b13_ref[...]
    he1 = _ln(he.astype(jnp.float32) + me, n3g_ref[...], n3b_ref[...])

    a = _bdot(hv, w1a_ref[...]) + b1_ref[...]
    m = _bdot(he1, w1b_ref[...])
    m = (m + g[:, C:]).reshape(BLK, K, C) + a[:, None, :]
    m = _gelu(m.astype(jnp.bfloat16)).reshape(BLK * K, C)
    m = jnp.dot(m, w2_ref[...], preferred_element_type=jnp.float32)
    m = _gelu(m.astype(jnp.bfloat16) + b2_ref[...])
    m = jnp.dot(m, w3_ref[...],
                preferred_element_type=jnp.float32) + b3_ref[...]
    m = m * ma_ref[...]
    dh = jnp.sum(m.reshape(BLK, K, C), axis=1) * (1.0 / SCALE)
    h = _ln(hv + dh, n1g_ref[...], n1b_ref[...])
    t = _bdot(h, wf1_ref[...])
    t = _gelu(t.astype(jnp.bfloat16) + bf1_ref[...])
    dh2 = jnp.dot(t, wf2_ref[...],
                  preferred_element_type=jnp.float32) + bf2_ref[...]
    h = _ln(h + dh2, n2g_ref[...], n2b_ref[...])
    hv2_ref[...] = h * mask_ref[...]


def _full_spec(shape):
    nd = len(shape)
    return pl.BlockSpec(shape, lambda i: (0,) * nd)


def _node_spec(d):
    return pl.BlockSpec((BLK, d), lambda i: (i, 0))


def _edge_spec(d):
    return pl.BlockSpec((BLK * K, d), lambda i: (i, 0))


def _run_pass1(hv0, he0, g0, ma, msk, ws):
    in_specs = ([_node_spec(C), _edge_spec(C), _edge_spec(C),
                 _edge_spec(1), _node_spec(1)]
                + [_full_spec(w.shape) for w in ws])
    return pl.pallas_call(
        _pass1_body,
        grid=(N // BLK,),
        in_specs=in_specs,
        out_specs=[_node_spec(C), _node_spec(2 * C)],
        out_shape=[jax.ShapeDtypeStruct((N, C), jnp.float32),
                   jax.ShapeDtypeStruct((N, 2 * C), jnp.float32)],
        compiler_params=pltpu.CompilerParams(
            dimension_semantics=("arbitrary",)),
    )(hv0, he0, g0, ma, msk, *ws)


def _run_pass2(hv1, he0, g1, ma, msk, ws):
    in_specs = ([_node_spec(C), _edge_spec(C), _edge_spec(2 * C),
                 _edge_spec(1), _node_spec(1)]
                + [_full_spec(w.shape) for w in ws])
    return pl.pallas_call(
        _pass2_body,
        grid=(N // BLK,),
        in_specs=in_specs,
        out_specs=_node_spec(C),
        out_shape=jax.ShapeDtypeStruct((N, C), jnp.float32),
        compiler_params=pltpu.CompilerParams(
            dimension_semantics=("arbitrary",)),
    )(hv1, he0, g1, ma, msk, *ws)


def kernel(h_V, h_E, E_idx, mask, mask_attend, params):
    b, n, c = h_V.shape
    k = E_idx.shape[2]
    hv0 = h_V.reshape(n, c)
    he0 = h_E.reshape(n * k, c).astype(jnp.bfloat16)
    pad = jnp.arange(_RPAD - n * k, dtype=jnp.int32) % n
    idx = jnp.concatenate([E_idx.reshape(n * k), pad])
    msk = mask.reshape(n, 1)
    ma = mask_attend.reshape(n * k, 1)
    p0, p1 = params

    def row(x):
        return x.reshape(1, -1)

    def bw(x):
        return x.astype(jnp.bfloat16)

    table0 = _project(hv0, bw(p0['W1'][2 * c:, :]))
    g0 = _sc_gather(table0, idx)
    wp = bw(jnp.concatenate([p0['W11'][2 * c:, :], p1['W1'][2 * c:, :]],
                            axis=1))
    ws1 = [bw(p0['W1'][:c, :]), bw(p0['W1'][c:2 * c, :]), row(p0['b1']),
           bw(p0['W2']), bw(row(p0['b2'])), bw(p0['W3']), row(p0['b3']),
           row(p0['n1g']), row(p0['n1b']),
           bw(p0['Wf1']), bw(row(p0['bf1'])), bw(p0['Wf2']), row(p0['bf2']),
           row(p0['n2g']), row(p0['n2b']), wp]
    hv1, t1 = _run_pass1(hv0, he0, g0, ma, msk, ws1)

    g1 = _sc_gather(t1, idx)
    ws2 = [bw(p0['W11'][:c, :]), bw(p0['W11'][c:2 * c, :]), row(p0['b11']),
           bw(p0['W12']), bw(row(p0['b12'])), bw(p0['W13']), row(p0['b13']),
           row(p0['n3g']), row(p0['n3b']),
           bw(p1['W1'][:c, :]), bw(p1['W1'][c:2 * c, :]), row(p1['b1']),
           bw(p1['W2']), bw(row(p1['b2'])), bw(p1['W3']), row(p1['b3']),
           row(p1['n1g']), row(p1['n1b']),
           bw(p1['Wf1']), bw(row(p1['bf1'])), bw(p1['Wf2']), row(p1['bf2']),
           row(p1['n2g']), row(p1['n2b'])]
    hv2 = _run_pass2(hv1, he0, g1, ma, msk, ws2)
    return hv2.reshape(b, n, c)
